# Initial kernel scaffold; baseline (speedup 1.0000x reference)
import jax
import jax.numpy as jnp
from jax.experimental import pallas as pl


def kernel(enc_output, mel_max_length, length_target, pitch_target, energy_target, params):
    raise NotImplementedError("write your pallas kernel here")


# speedup vs baseline: 38.2796x; 38.2796x over previous
import numpy as np
import jax
import jax.numpy as jnp
from jax.experimental import pallas as pl
from jax.experimental.pallas import tpu as pltpu

_D = 256
_NB = 256
_F = 256
_MIN_P, _MAX_P = 80.0, 800.0
_MIN_E, _MAX_E = 0.0, 100.0


def _ln(x, g, b):
    m = jnp.mean(x, axis=-1, keepdims=True)
    v = jnp.mean((x - m) ** 2, axis=-1, keepdims=True)
    return (x - m) * jax.lax.rsqrt(v + 1e-5) * g + b


def _conv3(x, w, b):
    z = jnp.zeros((1, x.shape[1]), x.dtype)
    xm = jnp.concatenate([z, x[:-1]], axis=0)
    xp = jnp.concatenate([x[1:], z], axis=0)
    y = jnp.dot(xm, w[0], preferred_element_type=jnp.float32)
    y = y + jnp.dot(x, w[1], preferred_element_type=jnp.float32)
    y = y + jnp.dot(xp, w[2], preferred_element_type=jnp.float32)
    return y + b


def _pred_body(x, w1, b1, g1, bn1, w2, b2, g2, bn2, wl, bl):
    h = _ln(jnp.maximum(_conv3(x, w1, b1), 0.0), g1, bn1)
    h = _ln(jnp.maximum(_conv3(h, w2, b2), 0.0), g2, bn2)
    return jnp.sum(h * wl, axis=1, keepdims=True) + bl


def _one_pred_kernel(x_ref, w1, b1, g1, bn1, w2, b2, g2, bn2, wl, bl, out_ref):
    out_ref[0] = _pred_body(
        x_ref[0], w1[...], b1[...], g1[...], bn1[...],
        w2[...], b2[...], g2[...], bn2[...], wl[...], bl[0, 0])


def _two_pred_kernel(x_ref,
                     pw1, pb1, pg1, pbn1, pw2, pb2, pg2, pbn2, pwl, pbl,
                     ew1, eb1, eg1, ebn1, ew2, eb2, eg2, ebn2, ewl, ebl,
                     pout_ref, eout_ref):
    x = x_ref[0]
    pout_ref[0] = _pred_body(
        x, pw1[...], pb1[...], pg1[...], pbn1[...],
        pw2[...], pb2[...], pg2[...], pbn2[...], pwl[...], pbl[0, 0])
    eout_ref[0] = _pred_body(
        x, ew1[...], eb1[...], eg1[...], ebn1[...],
        ew2[...], eb2[...], eg2[...], ebn2[...], ewl[...], ebl[0, 0])


def _lr_kernel(enc_ref, lt_ref, pt_ref, et_ref, psp_ref, esp_ref,
               pemb_ref, eemb_ref, lr_ref, out_ref, *, fb_size, t_in):
    fb = pl.program_id(1)
    enc = enc_ref[0]
    lt = lt_ref[0].astype(jnp.float32)
    ii = jax.lax.broadcasted_iota(jnp.int32, (t_in, t_in), 0)
    jj = jax.lax.broadcasted_iota(jnp.int32, (t_in, t_in), 1)
    tri = (ii <= jj).astype(jnp.float32)
    cum = jnp.dot(lt, tri, preferred_element_type=jnp.float32)
    f0 = (fb * fb_size).astype(jnp.float32)
    fcol = jax.lax.broadcasted_iota(jnp.int32, (fb_size, 1), 0).astype(jnp.float32) + f0
    le = (cum <= fcol).astype(jnp.float32)
    idx = jnp.minimum(jnp.sum(le, axis=1, keepdims=True), float(t_in - 1))
    jj2 = jax.lax.broadcasted_iota(jnp.int32, (fb_size, t_in), 1).astype(jnp.float32)
    oh = (idx == jj2).astype(jnp.float32)
    lr = jnp.dot(oh, enc, preferred_element_type=jnp.float32)
    total = jnp.sum(lt)
    mask = (fcol < total).astype(jnp.float32)
    lr = lr * mask
    lr_ref[0] = lr
    lane = jax.lax.broadcasted_iota(jnp.int32, (fb_size, _NB), 1).astype(jnp.float32)
    pv = jnp.log(pt_ref[0] + 1.0)
    pcnt = jnp.sum((psp_ref[...] < pv).astype(jnp.float32), axis=1, keepdims=True)
    pb = jnp.minimum(pcnt, float(_NB - 1))
    ohp = (pb == lane).astype(jnp.float32)
    pe = jnp.dot(ohp, pemb_ref[...], preferred_element_type=jnp.float32)
    ev = jnp.log(et_ref[0] + 1.0)
    ecnt = jnp.sum((esp_ref[...] < ev).astype(jnp.float32), axis=1, keepdims=True)
    eb = jnp.minimum(ecnt, float(_NB - 1))
    ohe = (eb == lane).astype(jnp.float32)
    ee = jnp.dot(ohe, eemb_ref[...], preferred_element_type=jnp.float32)
    out_ref[0] = lr + pe + ee


def _full(shape):
    return pl.BlockSpec(shape, lambda b, *_: tuple(0 for _ in shape))


def kernel(enc_output, mel_max_length, length_target, pitch_target,
           energy_target, params):
    B, T, D = enc_output.shape
    MEL = pitch_target.shape[1]
    FB = 1024
    NFB = MEL // FB

    pitch_space = jnp.linspace(np.log(_MIN_P + 1.0), np.log(_MAX_P + 2.0), _NB)
    energy_space = jnp.linspace(np.log(_MIN_E + 1.0), np.log(_MAX_E + 2.0), _NB)

    def prep(pre):
        p = params
        return (p[pre + '_w1'], p[pre + '_b1'].reshape(1, _F),
                p[pre + '_g1'].reshape(1, _F), p[pre + '_bn1'].reshape(1, _F),
                p[pre + '_w2'], p[pre + '_b2'].reshape(1, _F),
                p[pre + '_g2'].reshape(1, _F), p[pre + '_bn2'].reshape(1, _F),
                p[pre + '_wl'].reshape(1, _F), p[pre + '_bl'].reshape(1, 1))

    wspecs = [_full((3, _D, _F)), _full((1, _F)), _full((1, _F)), _full((1, _F)),
              _full((3, _F, _F)), _full((1, _F)), _full((1, _F)), _full((1, _F)),
              _full((1, _F)), _full((1, 1))]

    lt3 = length_target.astype(jnp.int32).reshape(B, 1, T)
    pt3 = pitch_target.reshape(B, MEL, 1)
    et3 = energy_target.reshape(B, MEL, 1)
    import functools
    lr_out, out = pl.pallas_call(
        functools.partial(_lr_kernel, fb_size=FB, t_in=T),
        grid=(B, NFB),
        in_specs=[
            pl.BlockSpec((1, T, D), lambda b, f: (b, 0, 0)),
            pl.BlockSpec((1, 1, T), lambda b, f: (b, 0, 0)),
            pl.BlockSpec((1, FB, 1), lambda b, f: (b, f, 0)),
            pl.BlockSpec((1, FB, 1), lambda b, f: (b, f, 0)),
            pl.BlockSpec((1, _NB), lambda b, f: (0, 0)),
            pl.BlockSpec((1, _NB), lambda b, f: (0, 0)),
            pl.BlockSpec((_NB, _D), lambda b, f: (0, 0)),
            pl.BlockSpec((_NB, _D), lambda b, f: (0, 0)),
        ],
        out_specs=[
            pl.BlockSpec((1, FB, D), lambda b, f: (b, f, 0)),
            pl.BlockSpec((1, FB, D), lambda b, f: (b, f, 0)),
        ],
        out_shape=[
            jax.ShapeDtypeStruct((B, MEL, D), jnp.float32),
            jax.ShapeDtypeStruct((B, MEL, D), jnp.float32),
        ],
    )(enc_output, lt3, pt3, et3,
      pitch_space.reshape(1, _NB), energy_space.reshape(1, _NB),
      params['pitch_emb'], params['energy_emb'])

    dur3 = pl.pallas_call(
        _one_pred_kernel,
        grid=(B,),
        in_specs=[pl.BlockSpec((1, T, D), lambda b: (b, 0, 0))] + wspecs,
        out_specs=pl.BlockSpec((1, T, 1), lambda b: (b, 0, 0)),
        out_shape=jax.ShapeDtypeStruct((B, T, 1), jnp.float32),
    )(enc_output, *prep('dur'))

    pitch3, energy3 = pl.pallas_call(
        _two_pred_kernel,
        grid=(B,),
        in_specs=[pl.BlockSpec((1, MEL, D), lambda b: (b, 0, 0))]
                 + wspecs + wspecs,
        out_specs=[pl.BlockSpec((1, MEL, 1), lambda b: (b, 0, 0)),
                   pl.BlockSpec((1, MEL, 1), lambda b: (b, 0, 0))],
        out_shape=[jax.ShapeDtypeStruct((B, MEL, 1), jnp.float32),
                   jax.ShapeDtypeStruct((B, MEL, 1), jnp.float32)],
    )(lr_out, *prep('pitch'), *prep('energy'))

    return (out, dur3.reshape(B, T), pitch3.reshape(B, MEL),
            energy3.reshape(B, MEL))
